# speedup vs baseline: 3.6760x; 3.6760x over previous
import functools

import jax
import jax.numpy as jnp
from jax import lax
from jax.experimental import pallas as pl
from jax.experimental.pallas import tpu as pltpu
from jax.experimental.pallas import tpu_sc as plsc

NN = 10000
DD = 128
EE = 320000

NSC = 2
NSUB = 16
NW = NSC * NSUB

EB = 128
NB = 2528
PB = NB // NW
EPAD = NB * EB

NPAD = 10240
RPS = NPAD // NSUB
RCH = 128
NCH = RPS // RCH

DCOL = 16

_sc_mesh = plsc.VectorSubcoreMesh(core_axis_name="c", subcore_axis_name="s")


def _deg_body(dstb_hbm, out_hbm, ones_v, idx_d, acc_sh, sem):
    c = lax.axis_index("c")
    s = lax.axis_index("s")
    w = c * NSUB + s

    @pl.loop(0, RCH)
    def _zero(r):
        ones_v[r, pl.ds(0, DCOL)] = jnp.zeros((DCOL,), jnp.float32)

    @pl.loop(0, NCH)
    def _clr(k):
        pltpu.sync_copy(ones_v, acc_sh.at[pl.ds(s * RPS + k * RCH, RCH)])

    @pl.loop(0, RCH)
    def _one(r):
        ones_v[r, pl.ds(0, DCOL)] = jnp.full((DCOL,), 1.0, jnp.float32)

    plsc.subcore_barrier()

    @pl.loop(0, PB)
    def _edges(i):
        b = w * PB + i
        pltpu.sync_copy(dstb_hbm.at[b], idx_d)
        pltpu.sync_copy(ones_v, acc_sh.at[idx_d], add=True)

    plsc.subcore_barrier()

    @pl.loop(0, NCH)
    def _out(k):
        r0 = s * RPS + k * RCH
        pltpu.sync_copy(acc_sh.at[pl.ds(r0, RCH)], out_hbm.at[c, pl.ds(r0, RCH)])


@jax.jit
def _deg_call(dstb):
    kern = pl.kernel(
        _deg_body,
        out_type=jax.ShapeDtypeStruct((NSC, NPAD, DCOL), jnp.float32),
        mesh=_sc_mesh,
        scratch_types=[
            pltpu.VMEM((RCH, DCOL), jnp.float32),
            pltpu.VMEM((EB,), jnp.int32),
            pltpu.VMEM_SHARED((NPAD, DCOL), jnp.float32),
            pltpu.SemaphoreType.DMA,
        ],
    )
    return kern(dstb)


def _msg_body(hw_hbm, srcb_hbm, dstb_hbm, out_hbm, rows_v, idx_s, idx_d, acc_sh, sem):
    c = lax.axis_index("c")
    s = lax.axis_index("s")
    w = c * NSUB + s

    @pl.loop(0, RCH)
    def _zr(r):
        @pl.loop(0, DD, step=16)
        def _zc(k):
            rows_v[r, pl.ds(k, 16)] = jnp.zeros((16,), jnp.float32)

    @pl.loop(0, NCH)
    def _clr(k):
        pltpu.sync_copy(rows_v, acc_sh.at[pl.ds(s * RPS + k * RCH, RCH)])

    plsc.subcore_barrier()

    @pl.loop(0, PB)
    def _edges(i):
        b = w * PB + i
        pltpu.sync_copy(srcb_hbm.at[b], idx_s)
        pltpu.sync_copy(dstb_hbm.at[b], idx_d)
        pltpu.async_copy(hw_hbm.at[idx_s], rows_v, sem).wait()
        pltpu.sync_copy(rows_v, acc_sh.at[idx_d], add=True)

    plsc.subcore_barrier()

    @pl.loop(0, NCH)
    def _out(k):
        r0 = s * RPS + k * RCH
        pltpu.sync_copy(acc_sh.at[pl.ds(r0, RCH)], out_hbm.at[c, pl.ds(r0, RCH)])


@jax.jit
def _msg_call(hw, srcb, dstb):
    kern = pl.kernel(
        _msg_body,
        out_type=jax.ShapeDtypeStruct((NSC, NPAD, DD), jnp.float32),
        mesh=_sc_mesh,
        scratch_types=[
            pltpu.VMEM((EB, DD), jnp.float32),
            pltpu.VMEM((EB,), jnp.int32),
            pltpu.VMEM((EB,), jnp.int32),
            pltpu.VMEM_SHARED((NPAD, DD), jnp.float32),
            pltpu.SemaphoreType.DMA,
        ],
    )
    return kern(hw, srcb, dstb)


RB = 1000
NG = NN // RB


def _mm1_body(x_ref, w_ref, o_ref):
    x = x_ref[...]
    h = x / (jnp.sum(jnp.abs(x), axis=1, keepdims=True) + 1e-12)
    o_ref[...] = jnp.dot(h, w_ref[...], preferred_element_type=jnp.float32)


@jax.jit
def _mm1_call(x, w):
    return pl.pallas_call(
        _mm1_body,
        grid=(NG,),
        in_specs=[
            pl.BlockSpec((RB, DD), lambda i: (i, 0)),
            pl.BlockSpec((DD, DD), lambda i: (0, 0)),
        ],
        out_specs=pl.BlockSpec((RB, DD), lambda i: (i, 0)),
        out_shape=jax.ShapeDtypeStruct((NN, DD), jnp.float32),
    )(x, w)


def _norm_scale_body(deg_ref, hw_ref, norm_ref, o_ref):
    deg = deg_ref[0, :, 0:1] + deg_ref[1, :, 0:1] + 1.0
    norm = lax.rsqrt(deg)
    normb = jnp.broadcast_to(norm, (RB, DD))
    norm_ref[...] = normb
    o_ref[...] = hw_ref[...] * normb


@jax.jit
def _norm_scale_call(degtab, hw_raw):
    return pl.pallas_call(
        _norm_scale_body,
        grid=(NG,),
        in_specs=[
            pl.BlockSpec((NSC, RB, DCOL), lambda i: (0, i, 0)),
            pl.BlockSpec((RB, DD), lambda i: (i, 0)),
        ],
        out_specs=[
            pl.BlockSpec((RB, DD), lambda i: (i, 0)),
            pl.BlockSpec((RB, DD), lambda i: (i, 0)),
        ],
        out_shape=[
            jax.ShapeDtypeStruct((NN, DD), jnp.float32),
            jax.ShapeDtypeStruct((NN, DD), jnp.float32),
        ],
    )(degtab, hw_raw)


def _combine_mm_body(p_ref, hw_ref, n_ref, b_ref, w_ref, o_ref):
    n = n_ref[...]
    t = (p_ref[0] + p_ref[1] + hw_ref[...]) * n + b_ref[...]
    t = jnp.maximum(t, 0.0)
    o_ref[...] = jnp.dot(t, w_ref[...], preferred_element_type=jnp.float32) * n


@jax.jit
def _combine_mm_call(ptab, hw, normb, b, w):
    return pl.pallas_call(
        _combine_mm_body,
        grid=(NG,),
        in_specs=[
            pl.BlockSpec((NSC, RB, DD), lambda i: (0, i, 0)),
            pl.BlockSpec((RB, DD), lambda i: (i, 0)),
            pl.BlockSpec((RB, DD), lambda i: (i, 0)),
            pl.BlockSpec((1, DD), lambda i: (0, 0)),
            pl.BlockSpec((DD, DD), lambda i: (0, 0)),
        ],
        out_specs=pl.BlockSpec((RB, DD), lambda i: (i, 0)),
        out_shape=jax.ShapeDtypeStruct((NN, DD), jnp.float32),
    )(ptab, hw, normb, b, w)


def _combine_final_body(p_ref, hw_ref, n_ref, b_ref, o_ref):
    t = (p_ref[0] + p_ref[1] + hw_ref[...]) * n_ref[...] + b_ref[...]
    o_ref[...] = jnp.maximum(t, 0.0)


@jax.jit
def _combine_final_call(ptab, hw, normb, b):
    return pl.pallas_call(
        _combine_final_body,
        grid=(NG,),
        in_specs=[
            pl.BlockSpec((NSC, RB, DD), lambda i: (0, i, 0)),
            pl.BlockSpec((RB, DD), lambda i: (i, 0)),
            pl.BlockSpec((RB, DD), lambda i: (i, 0)),
            pl.BlockSpec((1, DD), lambda i: (0, 0)),
        ],
        out_specs=pl.BlockSpec((RB, DD), lambda i: (i, 0)),
        out_shape=jax.ShapeDtypeStruct((NN, DD), jnp.float32),
    )(ptab, hw, normb, b)


def kernel(x, edge_index, W_enc0, b_enc0, W_enc1, b_enc1, W_dec0, b_dec0,
           W_dec1, b_dec1):
    src = edge_index[0].astype(jnp.int32)
    dst = edge_index[1].astype(jnp.int32)
    srcb = jnp.concatenate([src, jnp.zeros((EPAD - EE,), jnp.int32)]).reshape(NB, EB)
    dstb = jnp.concatenate([dst, jnp.full((EPAD - EE,), NN, jnp.int32)]).reshape(NB, EB)

    b_enc0 = b_enc0.reshape(1, DD)
    b_enc1 = b_enc1.reshape(1, DD)
    b_dec0 = b_dec0.reshape(1, DD)
    b_dec1 = b_dec1.reshape(1, DD)

    degtab = _deg_call(dstb)
    hw1_raw = _mm1_call(x, W_enc0)
    normb, hw1 = _norm_scale_call(degtab, hw1_raw)

    p1 = _msg_call(hw1, srcb, dstb)
    hw2 = _combine_mm_call(p1, hw1, normb, b_enc0, W_enc1)
    p2 = _msg_call(hw2, srcb, dstb)
    hw3 = _combine_mm_call(p2, hw2, normb, b_enc1, W_dec0)
    p3 = _msg_call(hw3, srcb, dstb)
    hw4 = _combine_mm_call(p3, hw3, normb, b_dec0, W_dec1)
    p4 = _msg_call(hw4, srcb, dstb)
    recon = _combine_final_call(p4, hw4, normb, b_dec1)
    return recon
